# Initial kernel scaffold; baseline (speedup 1.0000x reference)
import jax
import jax.numpy as jnp
from jax.experimental import pallas as pl


def kernel(inputs, router_W, router_b, integ_W, integ_b, codebooks):
    raise NotImplementedError("write your pallas kernel here")


# speedup vs baseline: 1.8634x; 1.8634x over previous
import jax
import jax.numpy as jnp
from jax.experimental import pallas as pl
from jax.experimental.pallas import tpu as pltpu

N_TOK = 8192
H = 768
K = 1024
NC = 4
KS = NC * K
BLK = 256

_BF = jnp.bfloat16
_DN = (((1,), (1,)), ((), ()))
_DN0 = (((1,), (0,)), ((), ()))


def _mm(a, b, dn=_DN0):
    return jax.lax.dot_general(a, b, dn, preferred_element_type=jnp.float32)


def _main_body(x_ref, cbs_ref, cb16_ref, rW_ref, rb_ref, iW_ref, ib_ref,
               out_ref, w_ref, c2_ref):
    pid = pl.program_id(0)

    @pl.when(pid == 0)
    def _():
        sq = cbs_ref[...]
        sq = sq * sq
        c2col = jnp.sum(sq, axis=1, keepdims=True)
        c2_ref[...] = jnp.broadcast_to(c2col.T, (8, KS))

    x = x_ref[...]
    xb = x.astype(_BF)
    x2 = jnp.sum(x * x, axis=1, keepdims=True)

    logits = _mm(xb, rW_ref[...].astype(_BF)) + rb_ref[...]
    m = jnp.max(logits, axis=1, keepdims=True)
    e = jnp.exp(logits - m)
    w = e / jnp.sum(e, axis=1, keepdims=True)
    w_ref[...] = w

    iota = jax.lax.broadcasted_iota(jnp.int32, (BLK, K), 1)
    combined = jnp.zeros((BLK, H), jnp.float32)
    for i in range(NC):
        cbi = cb16_ref[i * K:(i + 1) * K, :]
        s = _mm(xb, cbi, _DN)
        d = (x2 - 2.0 * s) + c2_ref[0:1, i * K:(i + 1) * K]
        mi = jnp.min(d, axis=1, keepdims=True)
        ids = jnp.where(d <= mi, iota, K)
        kmin = jnp.min(ids, axis=1, keepdims=True)
        onehot = (iota == kmin).astype(_BF)
        qi = _mm(onehot, cbi)
        combined = combined + w[:, i:i + 1] * qi
    out_ref[...] = _mm(combined.astype(_BF), iW_ref[...].astype(_BF)) \
        + ib_ref[...]


@jax.jit
def kernel(inputs, router_W, router_b, integ_W, integ_b, codebooks):
    flat = inputs.reshape(N_TOK, H)
    cbs = codebooks.reshape(KS, H)
    cb16 = cbs.astype(_BF)

    out, w = pl.pallas_call(
        _main_body,
        grid=(N_TOK // BLK,),
        in_specs=[
            pl.BlockSpec((BLK, H), lambda i: (i, 0)),
            pl.BlockSpec((KS, H), lambda i: (0, 0)),
            pl.BlockSpec((KS, H), lambda i: (0, 0)),
            pl.BlockSpec((H, NC), lambda i: (0, 0)),
            pl.BlockSpec((1, NC), lambda i: (0, 0)),
            pl.BlockSpec((H, H), lambda i: (0, 0)),
            pl.BlockSpec((1, H), lambda i: (0, 0)),
        ],
        out_specs=[
            pl.BlockSpec((BLK, H), lambda i: (i, 0)),
            pl.BlockSpec((BLK, NC), lambda i: (i, 0)),
        ],
        out_shape=[
            jax.ShapeDtypeStruct((N_TOK, H), jnp.float32),
            jax.ShapeDtypeStruct((N_TOK, NC), jnp.float32),
        ],
        scratch_shapes=[pltpu.VMEM((8, KS), jnp.float32)],
    )(flat, cbs, cb16, router_W, router_b.reshape(1, NC), integ_W,
      integ_b.reshape(1, H))

    return (out.reshape(inputs.shape),
            w.reshape(inputs.shape[0], inputs.shape[1], NC))
